# baseline (speedup 1.0000x reference)
import functools

import jax
import jax.numpy as jnp
from jax import lax
from jax.experimental import pallas as pl
from jax.experimental.pallas import tpu as pltpu
from jax.experimental.pallas import tpu_sc as plsc

D = 1024
LANES = 16
NC, NS = 2, 16
NW = NC * NS
CHUNK = 8
NPE = 4
NO = 8


def _sc_gather_add(x2d, idx, table):
    rows = x2d.shape[0]
    b_per_w = rows // NW
    n_chunks = b_per_w // CHUNK
    mesh = plsc.VectorSubcoreMesh(core_axis_name="c", subcore_axis_name="s")

    buf = lambda: pltpu.VMEM((CHUNK, D), jnp.float32)
    sem = pltpu.SemaphoreType.DMA
    @functools.partial(
        pl.kernel,
        mesh=mesh,
        out_type=jax.ShapeDtypeStruct((rows, D), jnp.float32),
        scratch_types=[pltpu.VMEM((b_per_w,), jnp.int32)]
        + [buf() for _ in range(NPE + NO)]
        + [sem] * (NPE + 2 * NO),
    )
    def k(table_hbm, idx_hbm, x_hbm, out_hbm, idx_v, *bufs_and_sems):
        pe_v = bufs_and_sems[:NPE]
        o_v = bufs_and_sems[NPE:NPE + NO]
        gsem = bufs_and_sems[NPE + NO:2 * NPE + NO]
        xsem = bufs_and_sems[2 * NPE + NO:2 * NPE + 2 * NO]
        osem = bufs_and_sems[2 * NPE + 2 * NO:]

        wid = lax.axis_index("s") * NC + lax.axis_index("c")
        base = wid * b_per_w
        pltpu.sync_copy(idx_hbm.at[pl.ds(base, b_per_w)], idx_v)

        def gather(c, b):
            return pltpu.make_async_copy(
                table_hbm.at[idx_v.at[pl.ds(c * CHUNK, CHUNK)]], pe_v[b], gsem[b]
            )

        def x_copy(c, s):
            return pltpu.make_async_copy(
                x_hbm.at[pl.ds(base + c * CHUNK, CHUNK)], o_v[s], xsem[s]
            )

        def out_copy(c, s):
            return pltpu.make_async_copy(
                o_v[s], out_hbm.at[pl.ds(base + c * CHUNK, CHUNK)], osem[s]
            )

        for c in range(NPE):
            gather(c, c).start()
        for c in range(NPE):
            x_copy(c, c).start()

        @pl.loop(0, n_chunks, step=NO)
        def _group(c0):
            for u in range(NO):
                c = c0 + u
                b = u % NPE
                s = u
                gather(c, b).wait()
                x_copy(c, s).wait()

                @pl.loop(0, CHUNK)
                def _row(r):
                    for j in range(D // LANES):
                        sl = (r, pl.ds(j * LANES, LANES))
                        plsc.addupdate(o_v[s].at[sl], pe_v[b][sl])

                out_copy(c, s).start()

                @pl.when(c + NPE < n_chunks)
                def _():
                    gather(c + NPE, b).start()

                @pl.when(c >= NO - NPE)
                def _():
                    out_copy(c - (NO - NPE), (s + NPE) % NO).wait()

                @pl.when(c + NPE < n_chunks)
                def _():
                    x_copy(c + NPE, (s + NPE) % NO).start()

        for u in range(NO - NPE, NO):
            out_copy(n_chunks - NO + u, u).wait()

    return k(table, idx, x2d)


def kernel(x, position_ids, pe_table):
    b, lp1, d = x.shape
    pos = jnp.concatenate(
        [jnp.zeros((b, 1), dtype=jnp.int32), position_ids.astype(jnp.int32)],
        axis=1,
    ).reshape(-1)
    x2d = x.reshape(b * lp1, d)
    out = _sc_gather_add(x2d, pos, pe_table)
    return out.reshape(b, lp1, d)


# speedup vs baseline: 1.0218x; 1.0218x over previous
import functools

import jax
import jax.numpy as jnp
from jax import lax
from jax.experimental import pallas as pl
from jax.experimental.pallas import tpu as pltpu
from jax.experimental.pallas import tpu_sc as plsc

D = 1024
LANES = 16
NC, NS = 2, 16
NW = NC * NS
CHUNK = 16
NBUF = 2


def _sc_gather_add(x2d, idx, table):
    rows = x2d.shape[0]
    b_per_w = rows // NW
    n_chunks = b_per_w // CHUNK
    mesh = plsc.VectorSubcoreMesh(core_axis_name="c", subcore_axis_name="s")

    buf = lambda: pltpu.VMEM((CHUNK, D), jnp.float32)
    @functools.partial(
        pl.kernel,
        mesh=mesh,
        out_type=jax.ShapeDtypeStruct((rows, D), jnp.float32),
        scratch_types=[
            pltpu.VMEM((b_per_w,), jnp.int32),
            buf(), buf(),
            buf(), buf(),
            buf(), buf(),
            pltpu.SemaphoreType.DMA, pltpu.SemaphoreType.DMA,
            pltpu.SemaphoreType.DMA, pltpu.SemaphoreType.DMA,
            pltpu.SemaphoreType.DMA, pltpu.SemaphoreType.DMA,
        ],
    )
    def k(table_hbm, idx_hbm, x_hbm, out_hbm, idx_v,
          pe0, pe1, xv0, xv1, ov0, ov1, g0, g1, xs0, xs1, os0, os1):
        pe_v, x_v, o_v = (pe0, pe1), (xv0, xv1), (ov0, ov1)
        gsem, xsem, osem = (g0, g1), (xs0, xs1), (os0, os1)

        wid = lax.axis_index("s") * NC + lax.axis_index("c")
        base = wid * b_per_w
        pltpu.sync_copy(idx_hbm.at[pl.ds(base, b_per_w)], idx_v)

        def start_fetch(c, b):
            pltpu.async_copy(
                table_hbm.at[idx_v.at[pl.ds(c * CHUNK, CHUNK)]], pe_v[b], gsem[b]
            )
            pltpu.async_copy(
                x_hbm.at[pl.ds(base + c * CHUNK, CHUNK)], x_v[b], xsem[b]
            )

        def wait_fetch(c, b):
            pltpu.make_async_copy(
                table_hbm.at[idx_v.at[pl.ds(c * CHUNK, CHUNK)]], pe_v[b], gsem[b]
            ).wait()
            pltpu.make_async_copy(
                x_hbm.at[pl.ds(base + c * CHUNK, CHUNK)], x_v[b], xsem[b]
            ).wait()

        def out_copy(c, b):
            return pltpu.make_async_copy(
                o_v[b], out_hbm.at[pl.ds(base + c * CHUNK, CHUNK)], osem[b]
            )

        for b in range(NBUF):
            start_fetch(b, b)

        @pl.loop(0, n_chunks, step=NBUF)
        def _pair(c0):
            for b in range(NBUF):
                c = c0 + b
                wait_fetch(c, b)

                @pl.when(c0 > 0)
                def _():
                    out_copy(c - NBUF, b).wait()

                @pl.loop(0, CHUNK)
                def _row(r):
                    for j in range(D // LANES):
                        sl = (r, pl.ds(j * LANES, LANES))
                        o_v[b][sl] = pe_v[b][sl] + x_v[b][sl]

                out_copy(c, b).start()

                @pl.when(c + NBUF < n_chunks)
                def _():
                    start_fetch(c + NBUF, b)

        for b in range(NBUF):
            out_copy(n_chunks - NBUF + b, b).wait()

    return k(table, idx, x2d)


def kernel(x, position_ids, pe_table):
    b, lp1, d = x.shape
    pos = jnp.concatenate(
        [jnp.zeros((b, 1), dtype=jnp.int32), position_ids.astype(jnp.int32)],
        axis=1,
    ).reshape(-1)
    x2d = x.reshape(b * lp1, d)
    out = _sc_gather_add(x2d, pos, pe_table)
    return out.reshape(b, lp1, d)
